# baseline (speedup 1.0000x reference)
import functools

import jax
import jax.numpy as jnp
from jax import lax
from jax.experimental import pallas as pl
from jax.experimental.pallas import tpu as pltpu
from jax.experimental.pallas import tpu_sc as plsc

_K = 100
_NEG = -70.0
_B, _V, _S = 16, 4096, 1024
_LN = 16
_NBINS = 256
_NW = 32
_JOBS = _B * (_S // _LN)
_JPW = _JOBS // _NW
_UNROLL = 4


def _key_of(x):
    i = plsc.bitcast(x, jnp.int32)
    m = lax.shift_right_arithmetic(i, 31)
    ui = i ^ (m | jnp.int32(-2147483648))
    return plsc.bitcast(ui, jnp.uint32)


def _sc_body(logits_hbm, out_hbm, x_v, hist_v):
    cid = lax.axis_index("c")
    sid = lax.axis_index("s")
    wid = sid * 2 + cid
    lanes = lax.iota(jnp.int32, _LN)
    ones_i = jnp.ones((_LN,), jnp.int32)

    def do_job(j, carry):
        job = j * _NW + wid
        b = job // (_S // _LN)
        s0 = (job % (_S // _LN)) * _LN
        pltpu.sync_copy(logits_hbm.at[b, :, pl.ds(s0, _LN)], x_v)

        if True:
            prefix = jnp.zeros((_LN,), jnp.uint32)
            rank = jnp.full((_LN,), _K, jnp.int32)

            for p, shift in enumerate((24, 16, 8, 0)):
                def zero_body(i, _):
                    for u in range(_UNROLL):
                        hist_v[i * _UNROLL + u] = jnp.zeros((_LN,), jnp.int32)
                    return 0
                lax.fori_loop(0, _NBINS // _UNROLL, zero_body, 0)

                sh = jnp.uint32(shift)
                hi_sh = jnp.uint32(shift + 8)
                pref_hi = prefix >> hi_sh

                def hist_body(i, _):
                    for u in range(_UNROLL):
                        xv = x_v[i * _UNROLL + u]
                        uk = _key_of(xv)
                        binv = ((uk >> sh) & jnp.uint32(0xFF)).astype(jnp.int32)
                        if p == 0:
                            plsc.addupdate_scatter(
                                hist_v, [binv, lanes], ones_i)
                        else:
                            act = (uk >> hi_sh) == pref_hi
                            plsc.addupdate_scatter(
                                hist_v, [binv, lanes], ones_i, mask=act)
                    return 0
                lax.fori_loop(0, _V // _UNROLL, hist_body, 0)

                def scan_body(i, c):
                    cum, digit, above = c
                    for u in range(_UNROLL):
                        r_bin = _NBINS - 1 - (i * _UNROLL + u)
                        h = hist_v[r_bin]
                        cum2 = cum + h
                        crossed = (cum < rank) & (cum2 >= rank)
                        digit = jnp.where(crossed, r_bin, digit)
                        above = jnp.where(crossed, cum, above)
                        cum = cum2
                    return (cum, digit, above)

                zero_v = jnp.zeros((_LN,), jnp.int32)
                _, digit, above = lax.fori_loop(
                    0, _NBINS // _UNROLL, scan_body, (zero_v, zero_v, zero_v))

                prefix = prefix | (digit.astype(jnp.uint32) << sh)
                rank = rank - above

            t_u, n_keep = prefix, rank

        def out_body(i, cnteq):
            for u in range(_UNROLL):
                v = i * _UNROLL + u
                xv = x_v[v]
                uk = _key_of(xv)
                gt = uk > t_u
                eq = uk == t_u
                keep = gt | (eq & (cnteq < n_keep))
                x_v[v] = jnp.where(keep, xv, jnp.float32(_NEG))
                cnteq = cnteq + jnp.where(eq, 1, 0)
            return cnteq
        lax.fori_loop(0, _V // _UNROLL, out_body,
                      jnp.zeros((_LN,), jnp.int32))

        pltpu.sync_copy(x_v, out_hbm.at[b, :, pl.ds(s0, _LN)])
        return carry

    lax.fori_loop(0, _JPW, do_job, 0)


@jax.jit
def _topk_mask_sc(logits):
    mesh = plsc.VectorSubcoreMesh(core_axis_name="c", subcore_axis_name="s")
    fn = functools.partial(
        pl.kernel,
        mesh=mesh,
        out_type=jax.ShapeDtypeStruct((_B, _V, _S), jnp.float32),
        scratch_types=[pltpu.VMEM((_V, _LN), jnp.float32),
                       pltpu.VMEM((_NBINS, _LN), jnp.int32)],
        compiler_params=pltpu.CompilerParams(use_tc_tiling_on_sc=False,
                                             needs_layout_passes=False),
    )(_sc_body)
    return fn(logits)


def kernel(logits, k):
    del k
    return _topk_mask_sc(logits)


# speedup vs baseline: 3.1928x; 3.1928x over previous
import functools

import jax
import jax.numpy as jnp
from jax import lax
from jax.experimental import pallas as pl
from jax.experimental.pallas import tpu as pltpu
from jax.experimental.pallas import tpu_sc as plsc

_K = 100
_NEG = -70.0
_B, _V, _S = 16, 4096, 1024
_LN = 16
_NBINS = 256
_NW = 32
_JOBS = _B * (_S // _LN)
_JPW = _JOBS // _NW
_UNROLL = 4


def _key_of(x):
    i = plsc.bitcast(x, jnp.int32)
    m = lax.shift_right_arithmetic(i, 31)
    ui = i ^ (m | jnp.int32(-2147483648))
    return plsc.bitcast(ui, jnp.uint32)


def _sc_body(logits_hbm, out_hbm, x_v, hist_v):
    cid = lax.axis_index("c")
    sid = lax.axis_index("s")
    wid = sid * 2 + cid
    lanes = lax.iota(jnp.int32, _LN)
    ones_i = jnp.ones((_LN,), jnp.int32)

    def do_job(j, carry):
        job = j * _NW + wid
        b = job // (_S // _LN)
        s0 = (job % (_S // _LN)) * _LN
        pltpu.sync_copy(logits_hbm.at[b, :, pl.ds(s0, _LN)], x_v)

        prefix = jnp.zeros((_LN,), jnp.uint32)
        rank = jnp.full((_LN,), _K, jnp.int32)

        for p, shift in enumerate((24, 16, 8, 0)):
            @plsc.parallel_loop(0, _NBINS, unroll=8)
            def _(i):
                hist_v[i] = jnp.zeros((_LN,), jnp.int32)

            sh = jnp.uint32(shift)
            hi_sh = jnp.uint32(shift + 8)
            pref_hi = prefix >> hi_sh

            @plsc.parallel_loop(0, _V, unroll=8)
            def _(v):
                uk = _key_of(x_v[v])
                binv = ((uk >> sh) & jnp.uint32(0xFF)).astype(jnp.int32)
                if p == 0:
                    plsc.addupdate_scatter(hist_v, [binv, lanes], ones_i)
                else:
                    act = (uk >> hi_sh) == pref_hi
                    plsc.addupdate_scatter(hist_v, [binv, lanes], ones_i,
                                           mask=act)

            zero_v = jnp.zeros((_LN,), jnp.int32)

            @plsc.parallel_loop(0, _NBINS, unroll=8,
                                carry=(zero_v, zero_v, zero_v))
            def scan_res(i, c):
                cum, digit, above = c
                r_bin = _NBINS - 1 - i
                h = hist_v[r_bin]
                cum2 = cum + h
                crossed = (cum < rank) & (cum2 >= rank)
                digit = jnp.where(crossed, r_bin, digit)
                above = jnp.where(crossed, cum, above)
                return (cum2, digit, above)

            _, digit, above = scan_res
            prefix = prefix | (digit.astype(jnp.uint32) << sh)
            rank = rank - above

        t_u, n_keep = prefix, rank

        @plsc.parallel_loop(0, _V, unroll=8,
                            carry=jnp.zeros((_LN,), jnp.int32))
        def _(v, cnteq):
            xv = x_v[v]
            uk = _key_of(xv)
            gt = uk > t_u
            eq = uk == t_u
            keep = gt | (eq & (cnteq < n_keep))
            x_v[v] = jnp.where(keep, xv, jnp.float32(_NEG))
            return cnteq + jnp.where(eq, 1, 0)

        pltpu.sync_copy(x_v, out_hbm.at[b, :, pl.ds(s0, _LN)])
        return carry

    lax.fori_loop(0, _JPW, do_job, 0)


@jax.jit
def _topk_mask_sc(logits):
    mesh = plsc.VectorSubcoreMesh(core_axis_name="c", subcore_axis_name="s")
    fn = functools.partial(
        pl.kernel,
        mesh=mesh,
        out_type=jax.ShapeDtypeStruct((_B, _V, _S), jnp.float32),
        scratch_types=[pltpu.VMEM((_V, _LN), jnp.float32),
                       pltpu.VMEM((_NBINS, _LN), jnp.int32)],
        compiler_params=pltpu.CompilerParams(use_tc_tiling_on_sc=False,
                                             needs_layout_passes=False),
    )(_sc_body)
    return fn(logits)


def kernel(logits, k):
    del k
    return _topk_mask_sc(logits)
